# baseline (speedup 1.0000x reference)
import jax
import jax.numpy as jnp
from jax.experimental import pallas as pl
from jax.experimental.pallas import tpu as pltpu

_BLOCK_NODES = 100


def _scale_body(ei_ref, aw_ref, x_ref, o_ref, w_ref):
    @pl.when(pl.program_id(0) == 0)
    def _():
        ei = ei_ref[...]
        num_l = aw_ref.shape[0]
        onehot = (ei == jax.lax.broadcasted_iota(jnp.int32, (ei.shape[0], num_l), 1))
        w_ref[...] = jax.lax.dot_general(
            onehot.astype(jnp.float32), aw_ref[...],
            (((1,), (0,)), ((), ())),
            preferred_element_type=jnp.float32)

    o_ref[...] = x_ref[...] * w_ref[...][None]


def kernel(node_input, affine_weight, expand_index):
    n, m, c = node_input.shape
    aw = affine_weight.reshape(affine_weight.shape[-2], c)
    ei = expand_index.astype(jnp.int32).reshape(m, 1)

    bn = _BLOCK_NODES
    grid = (n // bn,)
    return pl.pallas_call(
        _scale_body,
        grid=grid,
        in_specs=[
            pl.BlockSpec((m, 1), lambda i: (0, 0)),
            pl.BlockSpec(aw.shape, lambda i: (0, 0)),
            pl.BlockSpec((bn, m, c), lambda i: (i, 0, 0)),
        ],
        out_specs=pl.BlockSpec((bn, m, c), lambda i: (i, 0, 0)),
        out_shape=jax.ShapeDtypeStruct((n, m, c), jnp.float32),
        scratch_shapes=[pltpu.VMEM((m, c), jnp.float32)],
    )(ei, aw, node_input)
